# Initial kernel scaffold; baseline (speedup 1.0000x reference)
import jax
import jax.numpy as jnp
from jax.experimental import pallas as pl


def kernel(preds, ground_truth):
    raise NotImplementedError("write your pallas kernel here")


# speedup vs baseline: 2.1370x; 2.1370x over previous
import jax
import jax.numpy as jnp
from jax import lax
from jax.experimental import pallas as pl
from jax.experimental.pallas import tpu as pltpu
from jax.experimental.pallas import tpu_sc as plsc

EPS = 1e-6
B, NV, VS = 1024, 64, 256
ROW = NV * VS
AW = 16 * VS
NC, NS, L = 2, 16, 16
NW = NC * NS
BPW = B // NW


def _sc_body(preds_hbm, gt_hbm, out_hbm, pbuf, gbuf):
    wid = lax.axis_index("s") * NC + lax.axis_index("c")
    base = wid * BPW

    def do_row(j, carry):
        b = base + j
        pltpu.sync_copy(preds_hbm.at[b], pbuf)
        pltpu.sync_copy(gt_hbm.at[b, pl.ds(2 * AW, AW)], gbuf)

        def patch(i, c):
            o = pl.multiple_of(i * L, L)
            x = pbuf[pl.ds(o, L)]
            p1 = pbuf[pl.ds(AW + o, L)]
            p2 = pbuf[pl.ds(3 * AW + o, L)]
            g = gbuf[pl.ds(o, L)]
            c1 = (p1 - g) + EPS
            c2 = (p2 - g) - EPS
            pbuf[pl.ds(o, L)] = jnp.maximum(jnp.maximum(c1, c2), x)
            return c

        lax.fori_loop(0, AW // L, patch, 0, unroll=4)
        pltpu.sync_copy(pbuf, out_hbm.at[b])
        return carry

    lax.fori_loop(0, BPW, do_row, 0)


def kernel(preds, ground_truth):
    p2 = preds.reshape(B, ROW)
    g2 = ground_truth.reshape(B, ROW)
    call = pl.kernel(
        _sc_body,
        out_type=jax.ShapeDtypeStruct((B, ROW), jnp.float32),
        mesh=plsc.VectorSubcoreMesh(core_axis_name="c", subcore_axis_name="s"),
        scratch_types=[
            pltpu.VMEM((ROW,), jnp.float32),
            pltpu.VMEM((AW,), jnp.float32),
        ],
    )
    out = call(p2, g2)
    return out.reshape(B, NV, VS)


# speedup vs baseline: 2.5173x; 1.1780x over previous
import jax
import jax.numpy as jnp
from jax import lax
from jax.experimental import pallas as pl
from jax.experimental.pallas import tpu as pltpu
from jax.experimental.pallas import tpu_sc as plsc

EPS = 1e-6
B, NV, VS = 1024, 64, 256
ROW = NV * VS
AW = 16 * VS
NC, NS, L = 2, 16, 16
NW = NC * NS
BPW = B // NW
NBUF = 4


def _patch(pbuf, gbuf):
    def body(i, c):
        o = pl.multiple_of(i * L, L)
        x = pbuf[pl.ds(o, L)]
        p1 = pbuf[pl.ds(AW + o, L)]
        p2 = pbuf[pl.ds(3 * AW + o, L)]
        g = gbuf[pl.ds(o, L)]
        c1 = (p1 - g) + EPS
        c2 = (p2 - g) - EPS
        pbuf[pl.ds(o, L)] = jnp.maximum(jnp.maximum(c1, c2), x)
        return c

    lax.fori_loop(0, AW // L, body, 0, unroll=4)


def _sc_body(preds_hbm, gt_hbm, out_hbm, pbufs, gbufs, sin_p, sin_g, souts):
    wid = lax.axis_index("s") * NC + lax.axis_index("c")
    base = wid * BPW

    def start_in(j):
        d = j % NBUF
        ip = pltpu.async_copy(preds_hbm.at[base + j], pbufs.at[d], sin_p.at[d])
        ig = pltpu.async_copy(
            gt_hbm.at[base + j, pl.ds(2 * AW, AW)], gbufs.at[d], sin_g.at[d])
        return ip, ig

    in_d = {0: start_in(0)}
    out_d = {}
    for j in range(BPW):
        d = j % NBUF
        if j + 1 < BPW:
            if j + 1 >= NBUF:
                out_d[j + 1 - NBUF].wait()
            in_d[j + 1] = start_in(j + 1)
        ip, ig = in_d.pop(j)
        ip.wait()
        ig.wait()
        _patch(pbufs.at[d], gbufs.at[d])
        out_d[j] = pltpu.async_copy(pbufs.at[d], out_hbm.at[base + j], souts.at[d])
    for j in range(BPW - NBUF + 1, BPW):
        out_d[j].wait()


def kernel(preds, ground_truth):
    p2 = preds.reshape(B, ROW)
    g2 = ground_truth.reshape(B, ROW)
    call = pl.kernel(
        _sc_body,
        out_type=jax.ShapeDtypeStruct((B, ROW), jnp.float32),
        mesh=plsc.VectorSubcoreMesh(core_axis_name="c", subcore_axis_name="s"),
        scratch_types=[
            pltpu.VMEM((NBUF, ROW), jnp.float32),
            pltpu.VMEM((NBUF, AW), jnp.float32),
            pltpu.SemaphoreType.DMA((NBUF,)),
            pltpu.SemaphoreType.DMA((NBUF,)),
            pltpu.SemaphoreType.DMA((NBUF,)),
        ],
    )
    out = call(p2, g2)
    return out.reshape(B, NV, VS)


# speedup vs baseline: 7.0995x; 2.8203x over previous
import jax
import jax.numpy as jnp
from jax import lax
from jax.experimental import pallas as pl
from jax.experimental.pallas import tpu as pltpu
from jax.experimental.pallas import tpu_sc as plsc

EPS = 1e-6
B, NV, VS = 1024, 64, 256
ROW = NV * VS
AW = 16 * VS
NC, NS, L = 2, 16, 16
NW = NC * NS
BPW = B // NW
NBUF = 4


def _patch(pbuf, gbuf):
    def outer(a, co):
        def body(c, cc):
            o = pl.multiple_of(c * L, L)
            x = pbuf[a, pl.ds(o, L)]
            p1 = pbuf[a + 16, pl.ds(o, L)]
            p2 = pbuf[a + 48, pl.ds(o, L)]
            g = gbuf[a, pl.ds(o, L)]
            c1 = (p1 - g) + EPS
            c2 = (p2 - g) - EPS
            pbuf[a, pl.ds(o, L)] = jnp.maximum(jnp.maximum(c1, c2), x)
            return cc

        lax.fori_loop(0, VS // L, body, 0, unroll=4)
        return co

    lax.fori_loop(0, 16, outer, 0)


def _sc_body(preds_hbm, gt_hbm, out_hbm, pbufs, gbufs, sin_p, sin_g, souts):
    wid = lax.axis_index("s") * NC + lax.axis_index("c")
    base = wid * BPW

    def start_in(j):
        d = j % NBUF
        ip = pltpu.async_copy(preds_hbm.at[base + j], pbufs.at[d], sin_p.at[d])
        ig = pltpu.async_copy(
            gt_hbm.at[base + j, pl.ds(32, 16)], gbufs.at[d], sin_g.at[d])
        return ip, ig

    in_d = {0: start_in(0)}
    out_d = {}
    for j in range(BPW):
        d = j % NBUF
        if j + 1 < BPW:
            if j + 1 >= NBUF:
                out_d[j + 1 - NBUF].wait()
            in_d[j + 1] = start_in(j + 1)
        ip, ig = in_d.pop(j)
        ip.wait()
        ig.wait()
        _patch(pbufs.at[d], gbufs.at[d])
        out_d[j] = pltpu.async_copy(pbufs.at[d], out_hbm.at[base + j], souts.at[d])
    for j in range(BPW - NBUF + 1, BPW):
        out_d[j].wait()


def kernel(preds, ground_truth):
    call = pl.kernel(
        _sc_body,
        out_type=jax.ShapeDtypeStruct((B, NV, VS), jnp.float32),
        mesh=plsc.VectorSubcoreMesh(core_axis_name="c", subcore_axis_name="s"),
        compiler_params=pltpu.CompilerParams(use_tc_tiling_on_sc=True),
        scratch_types=[
            pltpu.VMEM((NBUF, NV, VS), jnp.float32),
            pltpu.VMEM((NBUF, 16, VS), jnp.float32),
            pltpu.SemaphoreType.DMA((NBUF,)),
            pltpu.SemaphoreType.DMA((NBUF,)),
            pltpu.SemaphoreType.DMA((NBUF,)),
        ],
    )
    return call(preds, ground_truth)
